# Initial kernel scaffold; baseline (speedup 1.0000x reference)
import jax
import jax.numpy as jnp
from jax.experimental import pallas as pl


def kernel(x, F):
    raise NotImplementedError("write your pallas kernel here")


# speedup vs baseline: 7.1254x; 7.1254x over previous
import functools

import jax
import jax.numpy as jnp
from jax import lax
from jax.experimental import pallas as pl
from jax.experimental.pallas import tpu as pltpu
from jax.experimental.pallas import tpu_sc as plsc

B, L, D = 4096, 50, 64
NC, NS, LANES = 2, 16, 16
NW = NC * NS
BPW = B // NW

_mesh = plsc.VectorSubcoreMesh(core_axis_name="c", subcore_axis_name="s")


@functools.partial(
    pl.kernel,
    out_type=jax.ShapeDtypeStruct((B, D), jnp.float32),
    mesh=_mesh,
    scratch_types=[
        pltpu.VMEM((L, BPW), jnp.int32),
        pltpu.VMEM((BPW, D), jnp.float32),
        pltpu.VMEM((BPW,), jnp.int32),
        pltpu.VMEM_SHARED((NS * BPW, D), jnp.float32),
        pltpu.SemaphoreType.DMA,
    ],
    compiler_params=pltpu.CompilerParams(use_tc_tiling_on_sc=False),
)
def _feature_sc(xT_hbm, f_hbm, out_hbm, idx_v, buf, dst_v, acc_sh, sem):
    c = lax.axis_index("c")
    s = lax.axis_index("s")
    wid = c * NS + s
    base = wid * BPW
    region = s * BPW

    pltpu.sync_copy(xT_hbm.at[:, pl.ds(base, BPW)], idx_v)

    for k in range(BPW // LANES):
        dst_v[pl.ds(k * LANES, LANES)] = (
            lax.iota(jnp.int32, LANES) + (region + k * LANES)
        )

    pltpu.async_copy(f_hbm.at[idx_v.at[0]], buf, sem).wait()
    pltpu.sync_copy(buf, acc_sh.at[pl.ds(region, BPW)])

    def body(j, carry):
        pltpu.async_copy(f_hbm.at[idx_v.at[j]], buf, sem).wait()
        pltpu.sync_copy(buf, acc_sh.at[dst_v], add=True)
        return carry

    lax.fori_loop(1, L, body, 0)

    pltpu.sync_copy(acc_sh.at[pl.ds(region, BPW)], out_hbm.at[pl.ds(base, BPW)])


def kernel(x, F):
    xT = jnp.transpose(x.astype(jnp.int32))
    return _feature_sc(xT, F)


# speedup vs baseline: 9.4171x; 1.3216x over previous
import functools

import jax
import jax.numpy as jnp
from jax import lax
from jax.experimental import pallas as pl
from jax.experimental.pallas import tpu as pltpu
from jax.experimental.pallas import tpu_sc as plsc

B, L, D = 4096, 50, 64
NC, NS, LANES = 2, 16, 16
NW = NC * NS
BPW = B // NW
KJ = 5
NR = L // KJ
ROWS = KJ * BPW

_mesh = plsc.VectorSubcoreMesh(core_axis_name="c", subcore_axis_name="s")


@functools.partial(
    pl.kernel,
    out_type=jax.ShapeDtypeStruct((B, D), jnp.float32),
    mesh=_mesh,
    scratch_types=[
        pltpu.VMEM((L * BPW,), jnp.int32),
        pltpu.VMEM((ROWS, D), jnp.float32),
        pltpu.VMEM((ROWS, D), jnp.float32),
        pltpu.VMEM((ROWS,), jnp.int32),
        pltpu.VMEM_SHARED((NS * BPW, D), jnp.float32),
        pltpu.SemaphoreType.DMA,
        pltpu.SemaphoreType.DMA,
        pltpu.SemaphoreType.DMA,
        pltpu.SemaphoreType.DMA,
    ],
    compiler_params=pltpu.CompilerParams(use_tc_tiling_on_sc=False),
)
def _feature_sc(xw_hbm, f_hbm, out_hbm, idx_v, buf_a, buf_b, dst_v, acc_sh,
                sg_a, sg_b, ss_a, ss_b):
    c = lax.axis_index("c")
    s = lax.axis_index("s")
    wid = c * NS + s
    base = wid * BPW
    region = s * BPW

    pltpu.sync_copy(xw_hbm.at[wid], idx_v)

    for t in range(ROWS // LANES):
        col = (t * LANES) % BPW
        dst_v[pl.ds(t * LANES, LANES)] = lax.iota(jnp.int32, LANES) + (region + col)

    def _zrow(i, carry):
        for k in range(D // LANES):
            buf_a[i, pl.ds(k * LANES, LANES)] = jnp.zeros((LANES,), jnp.float32)
        return carry

    lax.fori_loop(0, BPW, _zrow, 0)
    pltpu.sync_copy(buf_a.at[pl.ds(0, BPW)], acc_sh.at[pl.ds(region, BPW)])

    bufs = (buf_a, buf_b)
    sg = (sg_a, sg_b)
    ss = (ss_a, ss_b)

    gathers = {}
    scatters = {}
    gathers[0] = pltpu.async_copy(
        f_hbm.at[idx_v.at[pl.ds(0, ROWS)]], buf_a, sg[0])
    for r in range(NR):
        pb, nb = r % 2, (r + 1) % 2
        if r >= 1:
            scatters[r - 1].wait()
        if r + 1 < NR:
            gathers[r + 1] = pltpu.async_copy(
                f_hbm.at[idx_v.at[pl.ds((r + 1) * ROWS, ROWS)]], bufs[nb], sg[nb])
        gathers[r].wait()
        scatters[r] = pltpu.async_copy(
            bufs[pb], acc_sh.at[dst_v], ss[pb], add=True)
    scatters[NR - 1].wait()

    pltpu.sync_copy(acc_sh.at[pl.ds(region, BPW)], out_hbm.at[pl.ds(base, BPW)])


def kernel(x, F):
    xw = (x.astype(jnp.int32)
          .reshape(NW, BPW, L)
          .transpose(0, 2, 1)
          .reshape(NW, L * BPW))
    return _feature_sc(xw, F)
